# baseline (speedup 1.0000x reference)
import functools

import jax
import jax.numpy as jnp
from jax import lax
from jax.experimental import pallas as pl
from jax.experimental.pallas import tpu as pltpu
from jax.experimental.pallas import tpu_sc as plsc

B = 16384
VOCAB = 100000
H = 64
N_CAT = 3
N_CONT = 2
TRANS_DIM = 371

_NC = 2
_NS = 16
_NW = _NC * _NS
_LANES = 16
_BPW = B // _NW


def _sc_gather_body(table_hbm, type_hbm, idx_hbm, out_hbm,
                    t_v, i_v, g_v, rows_v, sem):
    wid = lax.axis_index("s") * _NC + lax.axis_index("c")
    base = wid * _BPW
    pltpu.sync_copy(type_hbm.at[pl.ds(base, _BPW)], t_v)
    pltpu.sync_copy(idx_hbm.at[pl.ds(base, _BPW)], i_v)
    for k in range(_BPW // _LANES):
        sl = pl.ds(k * _LANES, _LANES)
        t = t_v[sl]
        ix = i_v[sl]
        g_v[sl] = jnp.minimum(t, N_CAT - 1) * VOCAB + ix
    pltpu.async_copy(table_hbm.at[g_v], rows_v, sem).wait()
    pltpu.sync_copy(rows_v, out_hbm.at[pl.ds(base, _BPW)])


_sc_gather = functools.partial(
    pl.kernel,
    out_type=jax.ShapeDtypeStruct((B, H), jnp.float32),
    mesh=plsc.VectorSubcoreMesh(core_axis_name="c", subcore_axis_name="s"),
    scratch_types=[
        pltpu.VMEM((_BPW,), jnp.int32),
        pltpu.VMEM((_BPW,), jnp.int32),
        pltpu.VMEM((_BPW,), jnp.int32),
        pltpu.VMEM((_BPW, H), jnp.float32),
        pltpu.SemaphoreType.DMA,
    ],
    compiler_params=pltpu.CompilerParams(use_tc_tiling_on_sc=False),
)(_sc_gather_body)


_R = 512
_NBLK = B // _R


def _tc_merge_body(type_ref, contv_ref, tf_ref, catemb_ref,
                   contW_ref, contb_ref, transW_ref, transb_ref, out_ref):
    t = type_ref[0]
    tcol = t.reshape(_R, 1)
    v = contv_ref[0].reshape(_R, 1)

    trans = lax.dot_general(
        tf_ref[...], transW_ref[...],
        dimension_numbers=(((1,), (1,)), ((), ())),
        preferred_element_type=jnp.float32,
    ) + transb_ref[0]

    w0 = contW_ref[0:1, :]
    w1 = contW_ref[1:2, :]
    b0 = contb_ref[0:1, :]
    b1 = contb_ref[1:2, :]
    is0 = tcol <= N_CAT
    cont = jnp.where(is0, v * w0 + b0, v * w1 + b1)

    is_cat = tcol < N_CAT
    is_cont = tcol < N_CAT + N_CONT
    out_ref[...] = jnp.where(is_cat, catemb_ref[...],
                             jnp.where(is_cont, cont, trans))


_tc_merge = pl.pallas_call(
    _tc_merge_body,
    grid=(_NBLK,),
    in_specs=[
        pl.BlockSpec((1, 1, _R), lambda i: (i, 0, 0)),
        pl.BlockSpec((1, 1, _R), lambda i: (i, 0, 0)),
        pl.BlockSpec((_R, TRANS_DIM), lambda i: (i, 0)),
        pl.BlockSpec((_R, H), lambda i: (i, 0)),
        pl.BlockSpec((N_CONT, H), lambda i: (0, 0)),
        pl.BlockSpec((N_CONT, H), lambda i: (0, 0)),
        pl.BlockSpec((H, TRANS_DIM), lambda i: (0, 0)),
        pl.BlockSpec((1, H), lambda i: (0, 0)),
    ],
    out_specs=pl.BlockSpec((_R, H), lambda i: (i, 0)),
    out_shape=jax.ShapeDtypeStruct((B, H), jnp.float32),
)


def kernel(type_id, cat_index, cont_value, trans_feat, cat_tables,
           cont_W, cont_b, trans_W, trans_b):
    table_flat = cat_tables.reshape(N_CAT * VOCAB, H)
    cat_emb = _sc_gather(table_flat, type_id, cat_index)
    out = _tc_merge(
        type_id.reshape(_NBLK, 1, _R),
        cont_value.reshape(_NBLK, 1, _R),
        trans_feat,
        cat_emb,
        cont_W,
        cont_b,
        trans_W,
        trans_b.reshape(1, H),
    )
    return out


# speedup vs baseline: 1.1175x; 1.1175x over previous
import functools

import jax
import jax.numpy as jnp
from jax import lax
from jax.experimental import pallas as pl
from jax.experimental.pallas import tpu as pltpu
from jax.experimental.pallas import tpu_sc as plsc

B = 16384
VOCAB = 100000
H = 64
N_CAT = 3
N_CONT = 2
TRANS_DIM = 371

_NC = 2
_NS = 16
_NW = _NC * _NS
_LANES = 16
_BPW = B // _NW


def _sc_gather_body(table_hbm, type_hbm, idx_hbm, out_hbm,
                    t_v, i_v, g_v, rows_v, sem):
    wid = lax.axis_index("s") * _NC + lax.axis_index("c")
    base = wid * _BPW
    pltpu.sync_copy(type_hbm.at[pl.ds(base, _BPW)], t_v)
    pltpu.sync_copy(idx_hbm.at[pl.ds(base, _BPW)], i_v)
    for k in range(_BPW // _LANES):
        sl = pl.ds(k * _LANES, _LANES)
        t = t_v[sl]
        ix = i_v[sl]
        g_v[sl] = jnp.minimum(t, N_CAT - 1) * VOCAB + ix
    pltpu.async_copy(table_hbm.at[g_v], rows_v, sem).wait()
    pltpu.sync_copy(rows_v, out_hbm.at[pl.ds(base, _BPW)])


_sc_gather = functools.partial(
    pl.kernel,
    out_type=jax.ShapeDtypeStruct((B, H), jnp.float32),
    mesh=plsc.VectorSubcoreMesh(core_axis_name="c", subcore_axis_name="s"),
    scratch_types=[
        pltpu.VMEM((_BPW,), jnp.int32),
        pltpu.VMEM((_BPW,), jnp.int32),
        pltpu.VMEM((_BPW,), jnp.int32),
        pltpu.VMEM((_BPW, H), jnp.float32),
        pltpu.SemaphoreType.DMA,
    ],
    compiler_params=pltpu.CompilerParams(use_tc_tiling_on_sc=False),
)(_sc_gather_body)


_R = 512
_NBLK = B // _R


def _tc_merge_body(type_ref, contv_ref, tfT_ref, catemb_ref,
                   contWT_ref, contbT_ref, transWT_ref, transb_ref, out_ref):
    t = type_ref[0]
    v = contv_ref[0]

    trans = lax.dot_general(
        transWT_ref[...], tfT_ref[...],
        dimension_numbers=(((0,), (0,)), ((), ())),
        preferred_element_type=jnp.float32,
    ) + transb_ref[...]

    w0 = contWT_ref[:, 0:1]
    w1 = contWT_ref[:, 1:2]
    b0 = contbT_ref[:, 0:1]
    b1 = contbT_ref[:, 1:2]
    is0 = t <= N_CAT
    cont = jnp.where(is0, v * w0 + b0, v * w1 + b1)

    cat = jnp.transpose(catemb_ref[...])

    is_cat = t < N_CAT
    is_cont = t < N_CAT + N_CONT
    out_ref[...] = jnp.where(is_cat, cat, jnp.where(is_cont, cont, trans))


_tc_merge = pl.pallas_call(
    _tc_merge_body,
    grid=(_NBLK,),
    in_specs=[
        pl.BlockSpec((1, 1, _R), lambda i: (i, 0, 0)),
        pl.BlockSpec((1, 1, _R), lambda i: (i, 0, 0)),
        pl.BlockSpec((TRANS_DIM, _R), lambda i: (0, i)),
        pl.BlockSpec((_R, H), lambda i: (i, 0)),
        pl.BlockSpec((H, N_CONT), lambda i: (0, 0)),
        pl.BlockSpec((H, N_CONT), lambda i: (0, 0)),
        pl.BlockSpec((TRANS_DIM, H), lambda i: (0, 0)),
        pl.BlockSpec((H, 1), lambda i: (0, 0)),
    ],
    out_specs=pl.BlockSpec((H, _R), lambda i: (0, i)),
    out_shape=jax.ShapeDtypeStruct((H, B), jnp.float32),
)


def kernel(type_id, cat_index, cont_value, trans_feat, cat_tables,
           cont_W, cont_b, trans_W, trans_b):
    table_flat = cat_tables.reshape(N_CAT * VOCAB, H)
    cat_emb = _sc_gather(table_flat, type_id, cat_index)
    out_t = _tc_merge(
        type_id.reshape(_NBLK, 1, _R),
        cont_value.reshape(_NBLK, 1, _R),
        trans_feat.T,
        cat_emb,
        cont_W.T,
        cont_b.T,
        trans_W.T,
        trans_b.reshape(H, 1),
    )
    return out_t.T
